# baseline (speedup 1.0000x reference)
import functools

import jax
import jax.numpy as jnp
from jax import lax
from jax.experimental import pallas as pl
from jax.experimental.pallas import tpu as pltpu
from jax.experimental.pallas import tpu_sc as plsc

E = 320000
N = 10000
N_PAD = 10240
D = 128
NC = 2
NS = 16
NW = NC * NS
K = 80
EPT_A = E // NS
CPT_A = EPT_A // K
RPT = N_PAD // NS
EPT_C = E // NW
CPT_C = EPT_C // K
RB = 2000

_SC_PARAMS = pltpu.CompilerParams(needs_layout_passes=False)


def _phase_a(edge_attr, pk_idx, zrows, zcnt):
  mesh = plsc.VectorSubcoreMesh(core_axis_name="c", subcore_axis_name="s")

  @functools.partial(
      pl.kernel,
      out_type=[
          jax.ShapeDtypeStruct((N_PAD, D), jnp.float32),
          jax.ShapeDtypeStruct((N_PAD, D), jnp.float32),
          jax.ShapeDtypeStruct((N_PAD,), jnp.float32),
          jax.ShapeDtypeStruct((N_PAD,), jnp.float32),
      ],
      mesh=mesh,
      scratch_types=[
          pltpu.VMEM((K,), jnp.int32),
          pltpu.VMEM((K,), jnp.int32),
          pltpu.VMEM((K,), jnp.int32),
          pltpu.VMEM((K,), jnp.int32),
          pltpu.VMEM((K, D), jnp.float32),
          pltpu.VMEM((K, D), jnp.float32),
          pltpu.VMEM((K,), jnp.float32),
          pltpu.VMEM_SHARED((N_PAD, D), jnp.float32),
          pltpu.VMEM_SHARED((N_PAD,), jnp.float32),
          pltpu.SemaphoreType.DMA,
          pltpu.SemaphoreType.DMA,
          pltpu.SemaphoreType.DMA,
          pltpu.SemaphoreType.DMA,
      ],
      compiler_params=_SC_PARAMS,
  )
  def k(ea, pk, zr, zc, o_sd, o_ss, o_cd, o_cs,
        pk0, pk1, idx0, idx1, rows0, rows1, ones, acc, acc_c,
        sem_l0, sem_l1, sem_s0, sem_s1):
    c = lax.axis_index("c")
    s = lax.axis_index("s")
    r0 = s * RPT
    pltpu.sync_copy(zr, acc.at[pl.ds(r0, RPT)])
    pltpu.sync_copy(zc, acc_c.at[pl.ds(r0, RPT)])
    for j in range(K // 16):
      ones[pl.ds(j * 16, 16)] = jnp.ones((16,), jnp.float32)
    plsc.subcore_barrier()

    eb = s * EPT_A

    def load(chunk, pkb, rows, sem):
      pltpu.async_copy(pk.at[pl.ds(eb + chunk * K, K)], pkb, sem)
      pltpu.async_copy(ea.at[pl.ds(eb + chunk * K, K)], rows, sem)

    def wait_load(pkb, rows, sem):
      pltpu.make_async_copy(pk.at[pl.ds(eb, K)], pkb, sem).wait()
      pltpu.make_async_copy(ea.at[pl.ds(eb, K)], rows, sem).wait()

    def unpack(pkb, idx):
      for j in range(K // 16):
        v = pkb[pl.ds(j * 16, 16)]
        idx[pl.ds(j * 16, 16)] = jnp.where(
            c == 0, lax.shift_right_logical(v, 16), v & 0xFFFF)

    def scatter(idx, rows, sem):
      pltpu.async_copy(rows, acc.at[idx], sem, add=True)
      pltpu.async_copy(ones, acc_c.at[idx], sem, add=True)

    def wait_scatter(idx, rows, sem):
      pltpu.make_async_copy(rows, acc.at[idx], sem).wait()
      pltpu.make_async_copy(ones, acc_c.at[idx], sem).wait()

    load(0, pk0, rows0, sem_l0)
    load(1, pk1, rows1, sem_l1)

    n2 = CPT_A // 2

    def pair(i2, carry):
      a = 2 * i2
      wait_load(pk0, rows0, sem_l0)
      unpack(pk0, idx0)
      scatter(idx0, rows0, sem_s0)
      wait_load(pk1, rows1, sem_l1)
      unpack(pk1, idx1)
      scatter(idx1, rows1, sem_s1)

      @pl.when(i2 < n2 - 1)
      def _():
        wait_scatter(idx0, rows0, sem_s0)
        load(a + 2, pk0, rows0, sem_l0)
        wait_scatter(idx1, rows1, sem_s1)
        load(a + 3, pk1, rows1, sem_l1)

      return carry

    lax.fori_loop(0, n2, pair, 0)
    wait_scatter(idx0, rows0, sem_s0)
    wait_scatter(idx1, rows1, sem_s1)
    plsc.subcore_barrier()

    @pl.when(c == 0)
    def _():
      pltpu.sync_copy(acc.at[pl.ds(r0, RPT)], o_sd.at[pl.ds(r0, RPT)])
      pltpu.sync_copy(acc_c.at[pl.ds(r0, RPT)], o_cd.at[pl.ds(r0, RPT)])

    @pl.when(c == 1)
    def _():
      pltpu.sync_copy(acc.at[pl.ds(r0, RPT)], o_ss.at[pl.ds(r0, RPT)])
      pltpu.sync_copy(acc_c.at[pl.ds(r0, RPT)], o_cs.at[pl.ds(r0, RPT)])

  return k(edge_attr, pk_idx, zrows, zcnt)


def _dense_body(sd, ss, cd, cs, w0, b0r, w1, b1r, w2, b2r, o_na, o_a, o_c):
  cdv = cd[...]
  csv = cs[...]
  nd = jnp.where(cdv > 0, sd[...] / jnp.maximum(cdv, 1.0), 0.0)
  ns_ = jnp.where(csv > 0, ss[...] / jnp.maximum(csv, 1.0), 0.0)
  w0v = w0[...]
  dn = (((1,), (1,)), ((), ()))
  x = lax.dot_general(nd, w0v[:, :D], dn, preferred_element_type=jnp.float32)
  x = x + lax.dot_general(ns_, w0v[:, D:], dn,
                          preferred_element_type=jnp.float32)
  x = jnp.maximum(x + b0r[...], 0.0)
  w1v = w1[...]
  y = lax.dot_general(x, w1v[:, :D], dn, preferred_element_type=jnp.float32)
  y = y + lax.dot_general(ns_, w1v[:, D:], dn,
                          preferred_element_type=jnp.float32)
  y = jnp.maximum(y + b1r[...], 0.0)
  o_na[...] = y
  w2v = w2[...]
  o_a[...] = jnp.sum(y * w2v[:, :D], axis=1, keepdims=True) + b2r[0, 0]
  o_c[...] = jnp.sum(y * w2v[:, D:], axis=1, keepdims=True)


def _phase_b(sums_dst, sums_src, cnt_dst, cnt_src, W0, b0, W1, b1, W2, b2):
  grid = (N // RB,)
  full = lambda shape: pl.BlockSpec(shape, lambda i: (0,) * len(shape))
  return pl.pallas_call(
      _dense_body,
      grid=grid,
      in_specs=[
          pl.BlockSpec((RB, D), lambda i: (i, 0)),
          pl.BlockSpec((RB, D), lambda i: (i, 0)),
          pl.BlockSpec((RB, 1), lambda i: (i, 0)),
          pl.BlockSpec((RB, 1), lambda i: (i, 0)),
          full((D, 2 * D)),
          full((1, D)),
          full((D, 2 * D)),
          full((1, D)),
          full((1, 2 * D)),
          full((1, 1)),
      ],
      out_specs=[
          pl.BlockSpec((RB, D), lambda i: (i, 0)),
          pl.BlockSpec((RB, 1), lambda i: (i, 0)),
          pl.BlockSpec((RB, 1), lambda i: (i, 0)),
      ],
      out_shape=[
          jax.ShapeDtypeStruct((N, D), jnp.float32),
          jax.ShapeDtypeStruct((N, 1), jnp.float32),
          jax.ShapeDtypeStruct((N, 1), jnp.float32),
      ],
  )(sums_dst, sums_src, cnt_dst, cnt_src, W0, b0, W1, b1, W2, b2)


def _phase_c(na2, a_vec, c_vec, pk_idx):
  mesh = plsc.VectorSubcoreMesh(core_axis_name="c", subcore_axis_name="s")

  @functools.partial(
      pl.kernel,
      out_type=[
          jax.ShapeDtypeStruct((E, 2 * D), jnp.float32),
          jax.ShapeDtypeStruct((E,), jnp.float32),
      ],
      mesh=mesh,
      scratch_types=[
          pltpu.VMEM((K,), jnp.int32),
          pltpu.VMEM((K,), jnp.int32),
          pltpu.VMEM((K,), jnp.int32),
          pltpu.VMEM((K,), jnp.int32),
          pltpu.VMEM((K,), jnp.int32),
          pltpu.VMEM((K,), jnp.int32),
          pltpu.VMEM((K, D), jnp.float32),
          pltpu.VMEM((K, D), jnp.float32),
          pltpu.VMEM((K, D), jnp.float32),
          pltpu.VMEM((K, D), jnp.float32),
          pltpu.VMEM((K,), jnp.float32),
          pltpu.VMEM((K,), jnp.float32),
          pltpu.VMEM((N,), jnp.float32),
          pltpu.VMEM((N,), jnp.float32),
          pltpu.SemaphoreType.DMA,
          pltpu.SemaphoreType.DMA,
          pltpu.SemaphoreType.DMA,
          pltpu.SemaphoreType.DMA,
          pltpu.SemaphoreType.DMA,
          pltpu.SemaphoreType.DMA,
      ],
      compiler_params=_SC_PARAMS,
  )
  def k(nat, av, cv, pk, o_ee, o_lg,
        pk0, pk1, idxs0, idxd0, idxs1, idxd1, rs0, rd0, rs1, rd1,
        lg0, lg1, at, ct,
        sem_i0, sem_i1, sem_g0, sem_g1, sem_w0, sem_w1):
    c = lax.axis_index("c")
    s = lax.axis_index("s")
    wid = s * NC + c
    pltpu.sync_copy(av, at)
    pltpu.sync_copy(cv, ct)
    eb = wid * EPT_C

    def iload(chunk, pkb, sem):
      pltpu.async_copy(pk.at[pl.ds(eb + chunk * K, K)], pkb, sem)

    def wait_iload(pkb, sem):
      pltpu.make_async_copy(pk.at[pl.ds(eb, K)], pkb, sem).wait()

    def unpack(pkb, idxs, idxd):
      for j in range(K // 16):
        v = pkb[pl.ds(j * 16, 16)]
        idxs[pl.ds(j * 16, 16)] = v & 0xFFFF
        idxd[pl.ds(j * 16, 16)] = lax.shift_right_logical(v, 16)

    def gathers(idxs, idxd, rs, rd, sem):
      pltpu.async_copy(nat.at[idxs], rs, sem)
      pltpu.async_copy(nat.at[idxd], rd, sem)

    def wait_gathers(idxs, idxd, rs, rd, sem):
      pltpu.make_async_copy(nat.at[idxs], rs, sem).wait()
      pltpu.make_async_copy(nat.at[idxd], rd, sem).wait()

    def logits(idxs, idxd, lg):
      for j in range(K // 16):
        ivs = idxs[pl.ds(j * 16, 16)]
        ivd = idxd[pl.ds(j * 16, 16)]
        lg[pl.ds(j * 16, 16)] = (plsc.load_gather(at, [ivs]) +
                                 plsc.load_gather(ct, [ivd]))

    def writes(chunk, rs, rd, lg, sem):
      base = eb + chunk * K
      pltpu.async_copy(rs, o_ee.at[pl.ds(base, K), pl.ds(0, D)], sem)
      pltpu.async_copy(rd, o_ee.at[pl.ds(base, K), pl.ds(D, D)], sem)
      pltpu.async_copy(lg, o_lg.at[pl.ds(base, K)], sem)

    def wait_writes(rs, rd, lg, sem):
      pltpu.make_async_copy(rs, o_ee.at[pl.ds(0, K), pl.ds(0, D)], sem).wait()
      pltpu.make_async_copy(rd, o_ee.at[pl.ds(0, K), pl.ds(D, D)], sem).wait()
      pltpu.make_async_copy(lg, o_lg.at[pl.ds(0, K)], sem).wait()

    iload(0, pk0, sem_i0)
    iload(1, pk1, sem_i1)

    n2 = CPT_C // 2

    def pair(i2, carry):
      a = 2 * i2

      @pl.when(i2 > 0)
      def _():
        wait_writes(rs0, rd0, lg0, sem_w0)

      wait_iload(pk0, sem_i0)
      unpack(pk0, idxs0, idxd0)
      gathers(idxs0, idxd0, rs0, rd0, sem_g0)
      logits(idxs0, idxd0, lg0)
      wait_gathers(idxs0, idxd0, rs0, rd0, sem_g0)
      writes(a, rs0, rd0, lg0, sem_w0)
      iload(a + 2, pk0, sem_i0)

      @pl.when(i2 > 0)
      def _():
        wait_writes(rs1, rd1, lg1, sem_w1)

      wait_iload(pk1, sem_i1)
      unpack(pk1, idxs1, idxd1)
      gathers(idxs1, idxd1, rs1, rd1, sem_g1)
      logits(idxs1, idxd1, lg1)
      wait_gathers(idxs1, idxd1, rs1, rd1, sem_g1)
      writes(a + 1, rs1, rd1, lg1, sem_w1)

      @pl.when(i2 < n2 - 1)
      def _():
        iload(a + 3, pk1, sem_i1)

      return carry

    lax.fori_loop(0, n2, pair, 0)

    tail = CPT_C - 1
    wait_writes(rs0, rd0, lg0, sem_w0)
    wait_iload(pk0, sem_i0)
    unpack(pk0, idxs0, idxd0)
    gathers(idxs0, idxd0, rs0, rd0, sem_g0)
    logits(idxs0, idxd0, lg0)
    wait_gathers(idxs0, idxd0, rs0, rd0, sem_g0)
    writes(tail, rs0, rd0, lg0, sem_w0)
    wait_writes(rs1, rd1, lg1, sem_w1)
    wait_writes(rs0, rd0, lg0, sem_w0)

  return k(na2, a_vec, c_vec, pk_idx)


def kernel(edge_attr, edge_index, W0, b0, W1, b1, W2, b2):
  src_idx = edge_index[0].astype(jnp.int32)
  dst_idx = edge_index[1].astype(jnp.int32)
  pk_idx = src_idx | (dst_idx << 16)
  zrows = jnp.zeros((RPT, D), jnp.float32)
  zcnt = jnp.zeros((RPT,), jnp.float32)
  sums_dst, sums_src, cnt_dst, cnt_src = _phase_a(
      edge_attr, pk_idx, zrows, zcnt)
  na2, a_col, c_col = _phase_b(
      sums_dst, sums_src,
      cnt_dst.reshape(N_PAD, 1), cnt_src.reshape(N_PAD, 1),
      W0, b0.reshape(1, D), W1, b1.reshape(1, D),
      W2, b2.reshape(1, 1))
  edge_embs, logits = _phase_c(
      na2, a_col.reshape(N), c_col.reshape(N), pk_idx)
  return (logits, edge_embs, na2)


# speedup vs baseline: 1.2470x; 1.2470x over previous
import functools

import jax
import jax.numpy as jnp
from jax import lax
from jax.experimental import pallas as pl
from jax.experimental.pallas import tpu as pltpu
from jax.experimental.pallas import tpu_sc as plsc

E = 320000
N = 10000
N_PAD = 10240
D = 128
NC = 2
NS = 16
NW = NC * NS
K = 80
NB = 4
EPT_A = E // NS
CPT_A = EPT_A // K
RPT = N_PAD // NS
EPT_C = E // NW
CPT_C = EPT_C // K
RB = 2000

_SC_PARAMS = pltpu.CompilerParams(needs_layout_passes=False)


def _phase_a(edge_attr, pk_idx, zrows, zcnt):
  mesh = plsc.VectorSubcoreMesh(core_axis_name="c", subcore_axis_name="s")

  @functools.partial(
      pl.kernel,
      out_type=[
          jax.ShapeDtypeStruct((N_PAD, D), jnp.float32),
          jax.ShapeDtypeStruct((N_PAD, D), jnp.float32),
          jax.ShapeDtypeStruct((N_PAD,), jnp.float32),
          jax.ShapeDtypeStruct((N_PAD,), jnp.float32),
      ],
      mesh=mesh,
      scratch_types=(
          [pltpu.VMEM((K,), jnp.int32) for _ in range(NB)] +
          [pltpu.VMEM((K,), jnp.int32) for _ in range(NB)] +
          [pltpu.VMEM((K, D), jnp.float32) for _ in range(NB)] +
          [
              pltpu.VMEM((K,), jnp.float32),
              pltpu.VMEM_SHARED((N_PAD, D), jnp.float32),
              pltpu.VMEM_SHARED((N_PAD,), jnp.float32),
          ] +
          [pltpu.SemaphoreType.DMA for _ in range(NB)] +
          [pltpu.SemaphoreType.DMA for _ in range(NB)]
      ),
      compiler_params=_SC_PARAMS,
  )
  def k(ea, pk, zr, zc, o_sd, o_ss, o_cd, o_cs, *bufs):
    pkb = bufs[0:NB]
    idxb = bufs[NB:2 * NB]
    rowsb = bufs[2 * NB:3 * NB]
    ones, acc, acc_c = bufs[3 * NB:3 * NB + 3]
    sem_l = bufs[3 * NB + 3:4 * NB + 3]
    sem_s = bufs[4 * NB + 3:5 * NB + 3]
    c = lax.axis_index("c")
    s = lax.axis_index("s")
    r0 = s * RPT
    pltpu.sync_copy(zr, acc.at[pl.ds(r0, RPT)])
    pltpu.sync_copy(zc, acc_c.at[pl.ds(r0, RPT)])
    for j in range(K // 16):
      ones[pl.ds(j * 16, 16)] = jnp.ones((16,), jnp.float32)
    plsc.subcore_barrier()

    eb = s * EPT_A

    def load(chunk, p):
      pltpu.async_copy(pk.at[pl.ds(eb + chunk * K, K)], pkb[p], sem_l[p])
      pltpu.async_copy(ea.at[pl.ds(eb + chunk * K, K)], rowsb[p], sem_l[p])

    def wait_load(p):
      pltpu.make_async_copy(pk.at[pl.ds(eb, K)], pkb[p], sem_l[p]).wait()
      pltpu.make_async_copy(ea.at[pl.ds(eb, K)], rowsb[p], sem_l[p]).wait()

    def unpack(p):
      for j in range(K // 16):
        v = pkb[p][pl.ds(j * 16, 16)]
        idxb[p][pl.ds(j * 16, 16)] = jnp.where(
            c == 0, lax.shift_right_logical(v, 16), v & 0xFFFF)

    def scatter(p):
      pltpu.async_copy(rowsb[p], acc.at[idxb[p]], sem_s[p], add=True)
      pltpu.async_copy(ones, acc_c.at[idxb[p]], sem_s[p], add=True)

    def wait_scatter(p):
      pltpu.make_async_copy(rowsb[p], acc.at[idxb[p]], sem_s[p]).wait()
      pltpu.make_async_copy(ones, acc_c.at[idxb[p]], sem_s[p]).wait()

    for p in range(NB):
      load(p, p)

    nq = CPT_A // NB
    tail = CPT_A - NB * nq

    def quad(i4, carry):
      q = NB * i4
      for p in range(NB):
        wait_load(p)
        unpack(p)
        scatter(p)
      for p in range(NB):
        wait_scatter(p)

        @pl.when(q + NB + p < CPT_A)
        def _():
          load(q + NB + p, p)

      return carry

    lax.fori_loop(0, nq, quad, 0)
    for p in range(tail):
      wait_load(p)
      unpack(p)
      scatter(p)
    for p in range(tail):
      wait_scatter(p)
    plsc.subcore_barrier()

    @pl.when(c == 0)
    def _():
      pltpu.sync_copy(acc.at[pl.ds(r0, RPT)], o_sd.at[pl.ds(r0, RPT)])
      pltpu.sync_copy(acc_c.at[pl.ds(r0, RPT)], o_cd.at[pl.ds(r0, RPT)])

    @pl.when(c == 1)
    def _():
      pltpu.sync_copy(acc.at[pl.ds(r0, RPT)], o_ss.at[pl.ds(r0, RPT)])
      pltpu.sync_copy(acc_c.at[pl.ds(r0, RPT)], o_cs.at[pl.ds(r0, RPT)])

  return k(edge_attr, pk_idx, zrows, zcnt)


def _dense_body(sd, ss, cd, cs, w0, b0r, w1, b1r, w2, b2r, o_na, o_a, o_c):
  cdv = cd[...]
  csv = cs[...]
  nd = jnp.where(cdv > 0, sd[...] / jnp.maximum(cdv, 1.0), 0.0)
  ns_ = jnp.where(csv > 0, ss[...] / jnp.maximum(csv, 1.0), 0.0)
  w0v = w0[...]
  dn = (((1,), (1,)), ((), ()))
  x = lax.dot_general(nd, w0v[:, :D], dn, preferred_element_type=jnp.float32)
  x = x + lax.dot_general(ns_, w0v[:, D:], dn,
                          preferred_element_type=jnp.float32)
  x = jnp.maximum(x + b0r[...], 0.0)
  w1v = w1[...]
  y = lax.dot_general(x, w1v[:, :D], dn, preferred_element_type=jnp.float32)
  y = y + lax.dot_general(ns_, w1v[:, D:], dn,
                          preferred_element_type=jnp.float32)
  y = jnp.maximum(y + b1r[...], 0.0)
  o_na[...] = y
  w2v = w2[...]
  o_a[...] = jnp.sum(y * w2v[:, :D], axis=1, keepdims=True) + b2r[0, 0]
  o_c[...] = jnp.sum(y * w2v[:, D:], axis=1, keepdims=True)


def _phase_b(sums_dst, sums_src, cnt_dst, cnt_src, W0, b0, W1, b1, W2, b2):
  grid = (N // RB,)
  full = lambda shape: pl.BlockSpec(shape, lambda i: (0,) * len(shape))
  return pl.pallas_call(
      _dense_body,
      grid=grid,
      in_specs=[
          pl.BlockSpec((RB, D), lambda i: (i, 0)),
          pl.BlockSpec((RB, D), lambda i: (i, 0)),
          pl.BlockSpec((RB, 1), lambda i: (i, 0)),
          pl.BlockSpec((RB, 1), lambda i: (i, 0)),
          full((D, 2 * D)),
          full((1, D)),
          full((D, 2 * D)),
          full((1, D)),
          full((1, 2 * D)),
          full((1, 1)),
      ],
      out_specs=[
          pl.BlockSpec((RB, D), lambda i: (i, 0)),
          pl.BlockSpec((RB, 1), lambda i: (i, 0)),
          pl.BlockSpec((RB, 1), lambda i: (i, 0)),
      ],
      out_shape=[
          jax.ShapeDtypeStruct((N, D), jnp.float32),
          jax.ShapeDtypeStruct((N, 1), jnp.float32),
          jax.ShapeDtypeStruct((N, 1), jnp.float32),
      ],
  )(sums_dst, sums_src, cnt_dst, cnt_src, W0, b0, W1, b1, W2, b2)


def _phase_c(na2, a_vec, c_vec, pk_idx):
  mesh = plsc.VectorSubcoreMesh(core_axis_name="c", subcore_axis_name="s")

  @functools.partial(
      pl.kernel,
      out_type=[
          jax.ShapeDtypeStruct((E, 2 * D), jnp.float32),
          jax.ShapeDtypeStruct((E,), jnp.float32),
      ],
      mesh=mesh,
      scratch_types=(
          [pltpu.VMEM((K,), jnp.int32) for _ in range(NB)] +
          [pltpu.VMEM((K,), jnp.int32) for _ in range(NB)] +
          [pltpu.VMEM((K,), jnp.int32) for _ in range(NB)] +
          [pltpu.VMEM((K, D), jnp.float32) for _ in range(NB)] +
          [pltpu.VMEM((K, D), jnp.float32) for _ in range(NB)] +
          [pltpu.VMEM((K,), jnp.float32) for _ in range(NB)] +
          [
              pltpu.VMEM((N,), jnp.float32),
              pltpu.VMEM((N,), jnp.float32),
          ] +
          [pltpu.SemaphoreType.DMA for _ in range(NB)] +
          [pltpu.SemaphoreType.DMA for _ in range(NB)] +
          [pltpu.SemaphoreType.DMA for _ in range(NB)]
      ),
      compiler_params=_SC_PARAMS,
  )
  def k(nat, av, cv, pk, o_ee, o_lg, *bufs):
    pkb = bufs[0:NB]
    idxsb = bufs[NB:2 * NB]
    idxdb = bufs[2 * NB:3 * NB]
    rsb = bufs[3 * NB:4 * NB]
    rdb = bufs[4 * NB:5 * NB]
    lgb = bufs[5 * NB:6 * NB]
    at, ct = bufs[6 * NB:6 * NB + 2]
    sem_i = bufs[6 * NB + 2:7 * NB + 2]
    sem_g = bufs[7 * NB + 2:8 * NB + 2]
    sem_w = bufs[8 * NB + 2:9 * NB + 2]
    c = lax.axis_index("c")
    s = lax.axis_index("s")
    wid = s * NC + c
    pltpu.sync_copy(av, at)
    pltpu.sync_copy(cv, ct)
    eb = wid * EPT_C

    def iload(chunk, p):
      pltpu.async_copy(pk.at[pl.ds(eb + chunk * K, K)], pkb[p], sem_i[p])

    def wait_iload(p):
      pltpu.make_async_copy(pk.at[pl.ds(eb, K)], pkb[p], sem_i[p]).wait()

    def unpack(p):
      for j in range(K // 16):
        v = pkb[p][pl.ds(j * 16, 16)]
        idxsb[p][pl.ds(j * 16, 16)] = v & 0xFFFF
        idxdb[p][pl.ds(j * 16, 16)] = lax.shift_right_logical(v, 16)

    def gathers(p):
      pltpu.async_copy(nat.at[idxsb[p]], rsb[p], sem_g[p])
      pltpu.async_copy(nat.at[idxdb[p]], rdb[p], sem_g[p])

    def wait_gathers(p):
      pltpu.make_async_copy(nat.at[idxsb[p]], rsb[p], sem_g[p]).wait()
      pltpu.make_async_copy(nat.at[idxdb[p]], rdb[p], sem_g[p]).wait()

    def logits(p):
      for j in range(K // 16):
        ivs = idxsb[p][pl.ds(j * 16, 16)]
        ivd = idxdb[p][pl.ds(j * 16, 16)]
        lgb[p][pl.ds(j * 16, 16)] = (plsc.load_gather(at, [ivs]) +
                                     plsc.load_gather(ct, [ivd]))

    def writes(chunk, p):
      base = eb + chunk * K
      pltpu.async_copy(rsb[p], o_ee.at[pl.ds(base, K), pl.ds(0, D)], sem_w[p])
      pltpu.async_copy(rdb[p], o_ee.at[pl.ds(base, K), pl.ds(D, D)], sem_w[p])
      pltpu.async_copy(lgb[p], o_lg.at[pl.ds(base, K)], sem_w[p])

    def wait_writes(p):
      pltpu.make_async_copy(
          rsb[p], o_ee.at[pl.ds(0, K), pl.ds(0, D)], sem_w[p]).wait()
      pltpu.make_async_copy(
          rdb[p], o_ee.at[pl.ds(0, K), pl.ds(D, D)], sem_w[p]).wait()
      pltpu.make_async_copy(lgb[p], o_lg.at[pl.ds(0, K)], sem_w[p]).wait()

    for p in range(NB):
      iload(p, p)

    nq = CPT_C // NB
    tail = CPT_C - NB * nq

    def quad(i4, carry):
      q = NB * i4
      for p in range(NB):
        @pl.when(i4 > 0)
        def _():
          wait_writes(p)

        wait_iload(p)
        unpack(p)

        @pl.when(q + NB + p < CPT_C)
        def _():
          iload(q + NB + p, p)

        gathers(p)
      for p in range(NB):
        logits(p)
        wait_gathers(p)
        writes(q + p, p)
      return carry

    lax.fori_loop(0, nq, quad, 0)
    for p in range(tail):
      wait_writes(p)
      wait_iload(p)
      unpack(p)
      gathers(p)
      logits(p)
      wait_gathers(p)
      writes(NB * nq + p, p)
    for p in range(tail, NB):
      wait_writes(p)
    for p in range(tail):
      wait_writes(p)

  return k(na2, a_vec, c_vec, pk_idx)


def kernel(edge_attr, edge_index, W0, b0, W1, b1, W2, b2):
  src_idx = edge_index[0].astype(jnp.int32)
  dst_idx = edge_index[1].astype(jnp.int32)
  pk_idx = src_idx | (dst_idx << 16)
  zrows = jnp.zeros((RPT, D), jnp.float32)
  zcnt = jnp.zeros((RPT,), jnp.float32)
  sums_dst, sums_src, cnt_dst, cnt_src = _phase_a(
      edge_attr, pk_idx, zrows, zcnt)
  na2, a_col, c_col = _phase_b(
      sums_dst, sums_src,
      cnt_dst.reshape(N_PAD, 1), cnt_src.reshape(N_PAD, 1),
      W0, b0.reshape(1, D), W1, b1.reshape(1, D),
      W2, b2.reshape(1, 1))
  edge_embs, logits = _phase_c(
      na2, a_col.reshape(N), c_col.reshape(N), pk_idx)
  return (logits, edge_embs, na2)
